# Initial kernel scaffold; baseline (speedup 1.0000x reference)
import jax
import jax.numpy as jnp
from jax.experimental import pallas as pl


def kernel(node_emb, edge_index, W, b):
    raise NotImplementedError("write your pallas kernel here")


# speedup vs baseline: 4.6983x; 4.6983x over previous
import functools

import jax
import jax.numpy as jnp
from jax import lax
from jax.experimental import pallas as pl
from jax.experimental.pallas import tpu as pltpu
from jax.experimental.pallas import tpu_sc as plsc

N = 10000
E = 320000
D = 128
NP = 10240
CH = 125
CHP = 128
NCHT = E // CH
NCHW = NCHT // 32
EPT = E // 16
RPT = NP // 16
HALF = NP // 2
RPP = HALF // 16

_mesh = plsc.VectorSubcoreMesh(core_axis_name="c", subcore_axis_name="s")
_f32 = jnp.float32


def _zero_1d(ref, n):
    def body(i, _):
        ref[pl.ds(i * 16, 16)] = jnp.zeros((16,), _f32)
        return 0
    lax.fori_loop(0, n // 16, body, 0)


def _rsqrt16(d):
    bi = lax.bitcast_convert_type(d, jnp.int32)
    y = lax.bitcast_convert_type(
        jnp.int32(0x5F3759DF) - lax.shift_right_arithmetic(bi, 1), _f32)
    for _ in range(4):
        y = y * (1.5 - 0.5 * d * y * y)
    return jnp.where(d > 0.5, y, 0.0)


@functools.partial(
    pl.kernel,
    mesh=_mesh,
    compiler_params=pltpu.CompilerParams(needs_layout_passes=False),
    out_type=jax.ShapeDtypeStruct((NP,), _f32),
    scratch_types=[
        pltpu.VMEM((EPT,), jnp.int32),
        pltpu.VMEM((NP,), _f32),
        pltpu.VMEM((RPT,), _f32),
        pltpu.VMEM((RPT,), _f32),
        pltpu.VMEM((RPT,), _f32),
        pltpu.VMEM_SHARED((16, NP), _f32),
    ],
)
def _deg(dst_hbm, dinv_hbm, dst_v, deg_v, acc_v, tmp_v, dinv_v, shared):
    c = lax.axis_index("c")
    s = lax.axis_index("s")

    _zero_1d(deg_v, NP)
    pltpu.sync_copy(dst_hbm.at[pl.ds(s * EPT, EPT)], dst_v)
    ones = jnp.ones((16,), _f32)

    def deg_body(i, _):
        idx = dst_v[pl.ds(i * 16, 16)]
        plsc.addupdate_scatter(deg_v, [idx], ones)
        return 0
    lax.fori_loop(0, EPT // 16, deg_body, 0)

    pltpu.sync_copy(deg_v, shared.at[s])
    plsc.subcore_barrier()

    base = s * RPT
    _zero_1d(acc_v, RPT)
    for k in range(16):
        pltpu.sync_copy(shared.at[k, pl.ds(base, RPT)], tmp_v)

        def add_body(i, _):
            acc_v[pl.ds(i * 16, 16)] = (acc_v[pl.ds(i * 16, 16)]
                                        + tmp_v[pl.ds(i * 16, 16)])
            return 0
        lax.fori_loop(0, RPT // 16, add_body, 0)

    def rsqrt_body(i, _):
        dinv_v[pl.ds(i * 16, 16)] = _rsqrt16(acc_v[pl.ds(i * 16, 16)])
        return 0
    lax.fori_loop(0, RPT // 16, rsqrt_body, 0)

    @pl.when(c == 0)
    def _():
        pltpu.sync_copy(dinv_v, dinv_hbm.at[pl.ds(base, RPT)])


@functools.partial(
    pl.kernel,
    mesh=_mesh,
    out_type=jax.ShapeDtypeStruct((2, NP, D), _f32),
    scratch_types=[
        pltpu.VMEM((NCHW, CHP), jnp.int32),
        pltpu.VMEM((2, NCHW, CHP), jnp.int32),
        pltpu.VMEM((CHP, D), _f32),
        pltpu.VMEM_SHARED((HALF + 8, D), _f32),
        pltpu.SemaphoreType.DMA,
    ],
)
def _layer(z_hbm, src_hbm, dst_hbm, p_hbm, src_v, dst_v, rows_v, accum, gsem):
    c = lax.axis_index("c")
    s = lax.axis_index("s")
    w = c * 16 + s

    pltpu.sync_copy(src_hbm.at[pl.ds(w * NCHW, NCHW), :], src_v)
    pltpu.sync_copy(dst_hbm.at[:, pl.ds(w * NCHW, NCHW), :], dst_v)

    for p in range(2):
        def zrow(i, _):
            for v in range(D // 16):
                rows_v[i, pl.ds(v * 16, 16)] = jnp.zeros((16,), _f32)
            return 0
        lax.fori_loop(0, CHP, zrow, 0)
        pltpu.sync_copy(rows_v, accum.at[pl.ds(s * RPP, 128), :])
        pltpu.sync_copy(rows_v, accum.at[pl.ds(s * RPP + 128, 128), :])
        pltpu.sync_copy(rows_v.at[pl.ds(0, 64), :],
                        accum.at[pl.ds(s * RPP + 256, 64), :])
        plsc.subcore_barrier()

        def chunk_body(ci, _):
            pltpu.async_copy(z_hbm.at[src_v.at[ci]], rows_v, gsem).wait()
            pltpu.sync_copy(rows_v, accum.at[dst_v.at[p, ci]], add=True)
            return 0
        lax.fori_loop(0, NCHW, chunk_body, 0)
        plsc.subcore_barrier()

        pltpu.sync_copy(accum.at[pl.ds(s * RPP, RPP), :],
                        p_hbm.at[c, pl.ds(p * HALF + s * RPP, RPP), :])


def _scale_body(d_ref, x_ref, o_ref):
    o_ref[...] = d_ref[...] * x_ref[...]


def _scale(d, x):
    blk = 256
    row = pl.BlockSpec((blk, D), lambda i: (i, 0))
    dsp = pl.BlockSpec((blk, 1), lambda i: (i, 0))
    return pl.pallas_call(
        _scale_body,
        grid=(NP // blk,),
        in_specs=[dsp, row],
        out_specs=row,
        out_shape=jax.ShapeDtypeStruct((NP, D), _f32),
    )(d, x)


def _comb_body(d_ref, a_ref, b_ref, x1_ref, z2_ref):
    dd = d_ref[...]
    x1 = dd * (a_ref[...] + b_ref[...])
    x1_ref[...] = x1
    z2_ref[...] = dd * x1


def _comb(d, pa, pb):
    blk = 256
    row = pl.BlockSpec((blk, D), lambda i: (i, 0))
    dsp = pl.BlockSpec((blk, 1), lambda i: (i, 0))
    return pl.pallas_call(
        _comb_body,
        grid=(NP // blk,),
        in_specs=[dsp, row, row],
        out_specs=[row, row],
        out_shape=[jax.ShapeDtypeStruct((NP, D), _f32),
                   jax.ShapeDtypeStruct((NP, D), _f32)],
    )(d, pa, pb)


def _final_body(x_ref, x1_ref, a_ref, b_ref, d_ref, wt_ref, bias_ref, o_ref):
    x2 = d_ref[...] * (a_ref[...] + b_ref[...])
    acc = (x_ref[...] + x1_ref[...] + x2) * 0.3
    o_ref[...] = (jnp.dot(acc, wt_ref[...], preferred_element_type=_f32)
                  + bias_ref[...])


def _final(x, x1, pa, pb, d, wt, bias):
    blk = 256
    row = pl.BlockSpec((blk, D), lambda i: (i, 0))
    dsp = pl.BlockSpec((blk, 1), lambda i: (i, 0))
    return pl.pallas_call(
        _final_body,
        grid=(NP // blk,),
        in_specs=[row, row, row, row, dsp,
                  pl.BlockSpec((D, D), lambda i: (0, 0)),
                  pl.BlockSpec((1, D), lambda i: (0, 0))],
        out_specs=row,
        out_shape=jax.ShapeDtypeStruct((NP, D), _f32),
    )(x, x1, pa, pb, d, wt, bias)


def kernel(node_emb, edge_index, W, b):
    src = edge_index[0].astype(jnp.int32)
    dst = edge_index[1].astype(jnp.int32)
    srcp = jnp.pad(src.reshape(NCHT, CH), ((0, 0), (0, CHP - CH)))
    d0 = jnp.where(dst < HALF, dst, HALF)
    d1 = jnp.where(dst >= HALF, dst - HALF, HALF)
    dstp = jnp.pad(jnp.stack([d0, d1]).reshape(2, NCHT, CH),
                   ((0, 0), (0, 0), (0, CHP - CH)), constant_values=HALF)
    x_pad = jnp.pad(node_emb, ((0, NP - N), (0, 0)))

    dinv = _deg(dst)
    d2 = dinv.reshape(NP, 1)
    z = _scale(d2, x_pad)
    p1 = _layer(z, srcp, dstp)
    x1, z2 = _comb(d2, p1[0], p1[1])
    p2 = _layer(z2, srcp, dstp)
    out = _final(x_pad, x1, p2[0], p2[1], d2, W.T, b.reshape(1, D))
    return out[:N]
